# baseline (speedup 1.0000x reference)
import functools

import jax
import jax.numpy as jnp
from jax import lax
from jax.experimental import pallas as pl
from jax.experimental.pallas import tpu as pltpu

_NUM_LEVELS = 4
_K = 1024
_D = 256
_N = 16384
_BETA = 0.25
_T_BLK = 512
_SUB = 4
_T_SUB = _T_BLK // _SUB


def _rvq_body(z_ref, e0_ref, e1_ref, e2_ref, e3_ref,
              zq_ref, i0_ref, i1_ref, i2_ref, i3_ref,
              commit_ref, vq_ref, perp_ref,
              e2hi_s, ehi_s, emid_s, elo_s, embsq_s,
              counts_acc, commit_acc):
    i = pl.program_id(0)
    nblk = pl.num_programs(0)
    e_refs = (e0_ref, e1_ref, e2_ref, e3_ref)

    @pl.when(i == 0)
    def _init():
        counts_acc[...] = jnp.zeros_like(counts_acc)
        commit_acc[...] = jnp.zeros_like(commit_acc)
        for l in range(_NUM_LEVELS):
            e = e_refs[l][...]
            e_hi = e.astype(jnp.bfloat16)
            r1 = e - e_hi.astype(jnp.float32)
            e_mid = r1.astype(jnp.bfloat16)
            e_lo = (r1 - e_mid.astype(jnp.float32)).astype(jnp.bfloat16)
            e2hi_s[l] = jnp.float32(2.0).astype(jnp.bfloat16) * e_hi
            ehi_s[l] = e_hi
            emid_s[l] = e_mid
            elo_s[l] = e_lo
            embsq_s[pl.ds(l, 1), :] = jnp.sum(e * e, axis=1)[None, :]

    idx_refs = (i0_ref, i1_ref, i2_ref, i3_ref)
    lane = lax.broadcasted_iota(jnp.int32, (_T_SUB, _K), 1)
    dn_t = (((1,), (1,)), ((), ()))
    dn = (((1,), (0,)), ((), ()))

    commit_blk = jnp.zeros((1, 1), jnp.float32)
    counts_blk = jnp.zeros((1, _K), jnp.float32)
    ones_row = jnp.ones((1, _T_SUB), jnp.bfloat16)

    x0 = [z_ref[pl.ds(s * _T_SUB, _T_SUB), :] for s in range(_SUB)]
    resid = list(x0)
    qsum = [jnp.zeros_like(x0[0]) for _ in range(_SUB)]

    for l in range(_NUM_LEVELS):
        e2_hi = e2hi_s[l]
        embsq = embsq_s[pl.ds(l, 1), :]
        for s in range(_SUB):
            x = resid[s]
            xsq = jnp.sum(x * x, axis=1, keepdims=True)
            m2 = lax.dot_general(x.astype(jnp.bfloat16), e2_hi, dn_t,
                                 preferred_element_type=jnp.float32)
            d = (xsq + embsq) - m2
            dmin = jnp.min(d, axis=1, keepdims=True)
            idx = jnp.min(jnp.where(d == dmin, lane, _K), axis=1)
            idx_refs[l][pl.ds(s * _T_SUB, _T_SUB)] = idx.astype(jnp.int32)
            oh16 = (lane == idx[:, None]).astype(jnp.bfloat16)
            counts_blk = counts_blk + lax.dot_general(
                ones_row, oh16, dn,
                preferred_element_type=jnp.float32)
            q = (lax.dot_general(oh16, ehi_s[l], dn,
                                 preferred_element_type=jnp.float32)
                 + lax.dot_general(oh16, emid_s[l], dn,
                                   preferred_element_type=jnp.float32)
                 + lax.dot_general(oh16, elo_s[l], dn,
                                   preferred_element_type=jnp.float32))
            diff = q - x
            commit_blk = commit_blk + jnp.sum(diff * diff, axis=(0, 1),
                                              keepdims=True)
            q_st = x + diff
            qsum[s] = qsum[s] + q_st
            resid[s] = x - q_st

    for s in range(_SUB):
        zq_ref[pl.ds(s * _T_SUB, _T_SUB), :] = x0[s] + (qsum[s] - x0[s])
    counts_acc[...] += counts_blk
    commit_acc[...] += commit_blk

    @pl.when(i == nblk - 1)
    def _finalize():
        total = commit_acc[...] / jnp.float32(_N * _D)
        commit_ref[...] = total
        vq_ref[...] = jnp.float32(_BETA) * total
        counts = counts_acc[...]
        probs = counts / jnp.float32(_NUM_LEVELS * _N + 1e-10)
        ent_terms = jnp.where(probs > 0,
                              probs * jnp.log(probs + 1e-10),
                              jnp.zeros_like(probs))
        perp_ref[...] = jnp.exp(-jnp.sum(ent_terms, axis=1,
                                         keepdims=True))


@functools.partial(jax.jit, static_argnames=("interpret",))
def _rvq(z, emb0, emb1, emb2, emb3, interpret=False):
    nblk = _N // _T_BLK
    tok_spec = pl.BlockSpec((_T_BLK, _D), lambda i: (i, 0))
    emb_spec = pl.BlockSpec((_K, _D), lambda i: (0, 0))
    idx_spec = pl.BlockSpec((_T_BLK,), lambda i: (i,))
    scalar_spec = pl.BlockSpec((1, 1), lambda i: (0, 0))
    out = pl.pallas_call(
        _rvq_body,
        grid=(nblk,),
        in_specs=[tok_spec, emb_spec, emb_spec, emb_spec, emb_spec],
        out_specs=[tok_spec, idx_spec, idx_spec, idx_spec, idx_spec,
                   scalar_spec, scalar_spec, scalar_spec],
        out_shape=[
            jax.ShapeDtypeStruct((_N, _D), jnp.float32),
            jax.ShapeDtypeStruct((_N,), jnp.int32),
            jax.ShapeDtypeStruct((_N,), jnp.int32),
            jax.ShapeDtypeStruct((_N,), jnp.int32),
            jax.ShapeDtypeStruct((_N,), jnp.int32),
            jax.ShapeDtypeStruct((1, 1), jnp.float32),
            jax.ShapeDtypeStruct((1, 1), jnp.float32),
            jax.ShapeDtypeStruct((1, 1), jnp.float32),
        ],
        scratch_shapes=[
            pltpu.VMEM((_NUM_LEVELS, _K, _D), jnp.bfloat16),
            pltpu.VMEM((_NUM_LEVELS, _K, _D), jnp.bfloat16),
            pltpu.VMEM((_NUM_LEVELS, _K, _D), jnp.bfloat16),
            pltpu.VMEM((_NUM_LEVELS, _K, _D), jnp.bfloat16),
            pltpu.VMEM((8, _K), jnp.float32),
            pltpu.VMEM((1, _K), jnp.float32),
            pltpu.VMEM((1, 1), jnp.float32),
        ],
        interpret=interpret,
    )(z, emb0, emb1, emb2, emb3)
    zq, i0, i1, i2, i3, commit, vq, perp = out
    indices = jnp.stack([i0, i1, i2, i3], axis=-1)
    return (zq, indices, vq.reshape(()), commit.reshape(()),
            perp.reshape(()))


def kernel(z, emb0, emb1, emb2, emb3):
    return _rvq(z, emb0, emb1, emb2, emb3)


# speedup vs baseline: 1.7303x; 1.7303x over previous
import functools

import jax
import jax.numpy as jnp
from jax import lax
from jax.experimental import pallas as pl
from jax.experimental.pallas import tpu as pltpu

_NUM_LEVELS = 4
_K = 1024
_D = 256
_N = 16384
_BETA = 0.25
_T_BLK = 1024
_SUB = 2
_T_SUB = _T_BLK // _SUB


def _rvq_body(z_ref, e0_ref, e1_ref, e2_ref, e3_ref,
              zq_ref, i0_ref, i1_ref, i2_ref, i3_ref,
              commit_ref, vq_ref, perp_ref,
              e2hi_s, ehi_s, emid_s, elo_s, embsq_s,
              counts_acc, commit_acc):
    i = pl.program_id(0)
    nblk = pl.num_programs(0)
    e_refs = (e0_ref, e1_ref, e2_ref, e3_ref)

    @pl.when(i == 0)
    def _init():
        counts_acc[...] = jnp.zeros_like(counts_acc)
        commit_acc[...] = jnp.zeros_like(commit_acc)
        for l in range(_NUM_LEVELS):
            e = e_refs[l][...]
            e_hi = e.astype(jnp.bfloat16)
            r1 = e - e_hi.astype(jnp.float32)
            e_mid = r1.astype(jnp.bfloat16)
            e_lo = (r1 - e_mid.astype(jnp.float32)).astype(jnp.bfloat16)
            e2hi_s[l] = jnp.float32(2.0).astype(jnp.bfloat16) * e_hi
            ehi_s[l] = e_hi
            emid_s[l] = e_mid
            elo_s[l] = e_lo
            embsq_s[pl.ds(l, 1), :] = jnp.sum(e * e, axis=1)[None, :]

    idx_refs = (i0_ref, i1_ref, i2_ref, i3_ref)
    lane = lax.broadcasted_iota(jnp.int32, (_T_SUB, _K), 1)
    dn_t = (((1,), (1,)), ((), ()))
    dn = (((1,), (0,)), ((), ()))

    commit_blk = jnp.zeros((1, 1), jnp.float32)
    counts_blk = jnp.zeros((1, _K), jnp.float32)
    ones_row = jnp.ones((1, _T_SUB), jnp.bfloat16)

    x0 = [z_ref[pl.ds(s * _T_SUB, _T_SUB), :] for s in range(_SUB)]
    resid = list(x0)
    qsum = [jnp.zeros_like(x0[0]) for _ in range(_SUB)]

    for l in range(_NUM_LEVELS):
        e2_hi = e2hi_s[l]
        embsq = embsq_s[pl.ds(l, 1), :]
        for s in range(_SUB):
            x = resid[s]
            xsq = jnp.sum(x * x, axis=1, keepdims=True)
            m2 = lax.dot_general(x.astype(jnp.bfloat16), e2_hi, dn_t,
                                 preferred_element_type=jnp.float32)
            d = (xsq + embsq) - m2
            dmin = jnp.min(d, axis=1, keepdims=True)
            idx = jnp.min(jnp.where(d == dmin, lane, _K), axis=1)
            idx_refs[l][pl.ds(s * _T_SUB, _T_SUB)] = idx.astype(jnp.int32)
            oh16 = (lane == idx[:, None]).astype(jnp.bfloat16)
            counts_blk = counts_blk + lax.dot_general(
                ones_row, oh16, dn,
                preferred_element_type=jnp.float32)
            q = (lax.dot_general(oh16, ehi_s[l], dn,
                                 preferred_element_type=jnp.float32)
                 + lax.dot_general(oh16, emid_s[l], dn,
                                   preferred_element_type=jnp.float32)
                 + lax.dot_general(oh16, elo_s[l], dn,
                                   preferred_element_type=jnp.float32))
            diff = q - x
            commit_blk = commit_blk + jnp.sum(diff * diff, axis=(0, 1),
                                              keepdims=True)
            q_st = x + diff
            qsum[s] = qsum[s] + q_st
            resid[s] = x - q_st

    for s in range(_SUB):
        zq_ref[pl.ds(s * _T_SUB, _T_SUB), :] = x0[s] + (qsum[s] - x0[s])
    counts_acc[...] += counts_blk
    commit_acc[...] += commit_blk

    @pl.when(i == nblk - 1)
    def _finalize():
        total = commit_acc[...] / jnp.float32(_N * _D)
        commit_ref[...] = total
        vq_ref[...] = jnp.float32(_BETA) * total
        counts = counts_acc[...]
        probs = counts / jnp.float32(_NUM_LEVELS * _N + 1e-10)
        ent_terms = jnp.where(probs > 0,
                              probs * jnp.log(probs + 1e-10),
                              jnp.zeros_like(probs))
        perp_ref[...] = jnp.exp(-jnp.sum(ent_terms, axis=1,
                                         keepdims=True))


@functools.partial(jax.jit, static_argnames=("interpret",))
def _rvq(z, emb0, emb1, emb2, emb3, interpret=False):
    nblk = _N // _T_BLK
    tok_spec = pl.BlockSpec((_T_BLK, _D), lambda i: (i, 0))
    emb_spec = pl.BlockSpec((_K, _D), lambda i: (0, 0))
    idx_spec = pl.BlockSpec((_T_BLK,), lambda i: (i,))
    scalar_spec = pl.BlockSpec((1, 1), lambda i: (0, 0))
    out = pl.pallas_call(
        _rvq_body,
        grid=(nblk,),
        in_specs=[tok_spec, emb_spec, emb_spec, emb_spec, emb_spec],
        out_specs=[tok_spec, idx_spec, idx_spec, idx_spec, idx_spec,
                   scalar_spec, scalar_spec, scalar_spec],
        out_shape=[
            jax.ShapeDtypeStruct((_N, _D), jnp.float32),
            jax.ShapeDtypeStruct((_N,), jnp.int32),
            jax.ShapeDtypeStruct((_N,), jnp.int32),
            jax.ShapeDtypeStruct((_N,), jnp.int32),
            jax.ShapeDtypeStruct((_N,), jnp.int32),
            jax.ShapeDtypeStruct((1, 1), jnp.float32),
            jax.ShapeDtypeStruct((1, 1), jnp.float32),
            jax.ShapeDtypeStruct((1, 1), jnp.float32),
        ],
        scratch_shapes=[
            pltpu.VMEM((_NUM_LEVELS, _K, _D), jnp.bfloat16),
            pltpu.VMEM((_NUM_LEVELS, _K, _D), jnp.bfloat16),
            pltpu.VMEM((_NUM_LEVELS, _K, _D), jnp.bfloat16),
            pltpu.VMEM((_NUM_LEVELS, _K, _D), jnp.bfloat16),
            pltpu.VMEM((8, _K), jnp.float32),
            pltpu.VMEM((1, _K), jnp.float32),
            pltpu.VMEM((1, 1), jnp.float32),
        ],
        interpret=interpret,
    )(z, emb0, emb1, emb2, emb3)
    zq, i0, i1, i2, i3, commit, vq, perp = out
    indices = jnp.stack([i0, i1, i2, i3], axis=-1)
    return (zq, indices, vq.reshape(()), commit.reshape(()),
            perp.reshape(()))


def kernel(z, emb0, emb1, emb2, emb3):
    return _rvq(z, emb0, emb1, emb2, emb3)


# speedup vs baseline: 1.7753x; 1.0260x over previous
import functools

import jax
import jax.numpy as jnp
from jax import lax
from jax.experimental import pallas as pl
from jax.experimental.pallas import tpu as pltpu

_NUM_LEVELS = 4
_K = 1024
_D = 256
_N = 16384
_BETA = 0.25
_T_BLK = 2048
_SUB = 2
_T_SUB = _T_BLK // _SUB


def _rvq_body(z_ref, e0_ref, e1_ref, e2_ref, e3_ref,
              zq_ref, i0_ref, i1_ref, i2_ref, i3_ref,
              commit_ref, vq_ref, perp_ref,
              e2hi_s, ehi_s, emid_s, elo_s, embsq_s,
              counts_acc, commit_acc):
    i = pl.program_id(0)
    nblk = pl.num_programs(0)
    e_refs = (e0_ref, e1_ref, e2_ref, e3_ref)

    @pl.when(i == 0)
    def _init():
        counts_acc[...] = jnp.zeros_like(counts_acc)
        commit_acc[...] = jnp.zeros_like(commit_acc)
        for l in range(_NUM_LEVELS):
            e = e_refs[l][...]
            e_hi = e.astype(jnp.bfloat16)
            r1 = e - e_hi.astype(jnp.float32)
            e_mid = r1.astype(jnp.bfloat16)
            e_lo = (r1 - e_mid.astype(jnp.float32)).astype(jnp.bfloat16)
            e2hi_s[l] = jnp.float32(2.0).astype(jnp.bfloat16) * e_hi
            ehi_s[l] = e_hi
            emid_s[l] = e_mid
            elo_s[l] = e_lo
            embsq_s[pl.ds(l, 1), :] = jnp.sum(e * e, axis=1)[None, :]

    idx_refs = (i0_ref, i1_ref, i2_ref, i3_ref)
    lane = lax.broadcasted_iota(jnp.int32, (_T_SUB, _K), 1)
    dn_t = (((1,), (1,)), ((), ()))
    dn = (((1,), (0,)), ((), ()))

    commit_blk = jnp.zeros((1, 1), jnp.float32)
    counts_blk = jnp.zeros((1, _K), jnp.float32)
    ones_row = jnp.ones((1, _T_SUB), jnp.bfloat16)

    x0 = [z_ref[pl.ds(s * _T_SUB, _T_SUB), :] for s in range(_SUB)]
    resid = list(x0)
    qsum = [jnp.zeros_like(x0[0]) for _ in range(_SUB)]

    for l in range(_NUM_LEVELS):
        e2_hi = e2hi_s[l]
        embsq = embsq_s[pl.ds(l, 1), :]
        for s in range(_SUB):
            x = resid[s]
            xsq = jnp.sum(x * x, axis=1, keepdims=True)
            m2 = lax.dot_general(x.astype(jnp.bfloat16), e2_hi, dn_t,
                                 preferred_element_type=jnp.float32)
            d = (xsq + embsq) - m2
            dmin = jnp.min(d, axis=1, keepdims=True)
            idx = jnp.min(jnp.where(d == dmin, lane, _K), axis=1)
            idx_refs[l][pl.ds(s * _T_SUB, _T_SUB)] = idx.astype(jnp.int32)
            oh16 = (lane == idx[:, None]).astype(jnp.bfloat16)
            counts_blk = counts_blk + lax.dot_general(
                ones_row, oh16, dn,
                preferred_element_type=jnp.float32)
            q = (lax.dot_general(oh16, ehi_s[l], dn,
                                 preferred_element_type=jnp.float32)
                 + lax.dot_general(oh16, emid_s[l], dn,
                                   preferred_element_type=jnp.float32)
                 + lax.dot_general(oh16, elo_s[l], dn,
                                   preferred_element_type=jnp.float32))
            diff = q - x
            commit_blk = commit_blk + jnp.sum(diff * diff, axis=(0, 1),
                                              keepdims=True)
            q_st = x + diff
            qsum[s] = qsum[s] + q_st
            resid[s] = x - q_st

    for s in range(_SUB):
        zq_ref[pl.ds(s * _T_SUB, _T_SUB), :] = x0[s] + (qsum[s] - x0[s])
    counts_acc[...] += counts_blk
    commit_acc[...] += commit_blk

    @pl.when(i == nblk - 1)
    def _finalize():
        total = commit_acc[...] / jnp.float32(_N * _D)
        commit_ref[...] = total
        vq_ref[...] = jnp.float32(_BETA) * total
        counts = counts_acc[...]
        probs = counts / jnp.float32(_NUM_LEVELS * _N + 1e-10)
        ent_terms = jnp.where(probs > 0,
                              probs * jnp.log(probs + 1e-10),
                              jnp.zeros_like(probs))
        perp_ref[...] = jnp.exp(-jnp.sum(ent_terms, axis=1,
                                         keepdims=True))


@functools.partial(jax.jit, static_argnames=("interpret",))
def _rvq(z, emb0, emb1, emb2, emb3, interpret=False):
    nblk = _N // _T_BLK
    tok_spec = pl.BlockSpec((_T_BLK, _D), lambda i: (i, 0))
    emb_spec = pl.BlockSpec((_K, _D), lambda i: (0, 0))
    idx_spec = pl.BlockSpec((_T_BLK,), lambda i: (i,))
    scalar_spec = pl.BlockSpec((1, 1), lambda i: (0, 0))
    out = pl.pallas_call(
        _rvq_body,
        grid=(nblk,),
        in_specs=[tok_spec, emb_spec, emb_spec, emb_spec, emb_spec],
        out_specs=[tok_spec, idx_spec, idx_spec, idx_spec, idx_spec,
                   scalar_spec, scalar_spec, scalar_spec],
        out_shape=[
            jax.ShapeDtypeStruct((_N, _D), jnp.float32),
            jax.ShapeDtypeStruct((_N,), jnp.int32),
            jax.ShapeDtypeStruct((_N,), jnp.int32),
            jax.ShapeDtypeStruct((_N,), jnp.int32),
            jax.ShapeDtypeStruct((_N,), jnp.int32),
            jax.ShapeDtypeStruct((1, 1), jnp.float32),
            jax.ShapeDtypeStruct((1, 1), jnp.float32),
            jax.ShapeDtypeStruct((1, 1), jnp.float32),
        ],
        scratch_shapes=[
            pltpu.VMEM((_NUM_LEVELS, _K, _D), jnp.bfloat16),
            pltpu.VMEM((_NUM_LEVELS, _K, _D), jnp.bfloat16),
            pltpu.VMEM((_NUM_LEVELS, _K, _D), jnp.bfloat16),
            pltpu.VMEM((_NUM_LEVELS, _K, _D), jnp.bfloat16),
            pltpu.VMEM((8, _K), jnp.float32),
            pltpu.VMEM((1, _K), jnp.float32),
            pltpu.VMEM((1, 1), jnp.float32),
        ],
        interpret=interpret,
    )(z, emb0, emb1, emb2, emb3)
    zq, i0, i1, i2, i3, commit, vq, perp = out
    indices = jnp.stack([i0, i1, i2, i3], axis=-1)
    return (zq, indices, vq.reshape(()), commit.reshape(()),
            perp.reshape(()))


def kernel(z, emb0, emb1, emb2, emb3):
    return _rvq(z, emb0, emb1, emb2, emb3)
